# baseline (speedup 1.0000x reference)
import jax
import jax.numpy as jnp
from jax.experimental import pallas as pl
from jax.experimental.pallas import tpu as pltpu

B = 64
P = 576
D = 384
BB = 16
PB = 288


def _add_body(pos_ref, table_ref, patches_ref, out_ref, emb_ref):
    @pl.when(pl.program_id(0) + pl.program_id(1) == 0)
    def _():
        pos = pos_ref[...]
        onehot = (pos[:, None] == jax.lax.broadcasted_iota(jnp.int32, (P, P), 1))
        emb_ref[...] = jnp.dot(onehot.astype(jnp.float32), table_ref[...],
                               preferred_element_type=jnp.float32)

    j = pl.program_id(1)
    out_ref[...] = patches_ref[...] + emb_ref[pl.ds(j * PB, PB), :][None]


_tc_kernel = pl.pallas_call(
    _add_body,
    grid=(B // BB, P // PB),
    in_specs=[
        pl.BlockSpec((P,), lambda i, j: (0,)),
        pl.BlockSpec((P, D), lambda i, j: (0, 0)),
        pl.BlockSpec((BB, PB, D), lambda i, j: (i, j, 0)),
    ],
    out_specs=pl.BlockSpec((BB, PB, D), lambda i, j: (i, j, 0)),
    out_shape=jax.ShapeDtypeStruct((B, P, D), jnp.float32),
    scratch_shapes=[pltpu.VMEM((P, D), jnp.float32)],
)


def kernel(encoded_patches, pos_embedding_table, positions):
    return _tc_kernel(positions, pos_embedding_table, encoded_patches)


# speedup vs baseline: 1.0175x; 1.0175x over previous
import jax
import jax.numpy as jnp
from jax.experimental import pallas as pl
from jax.experimental.pallas import tpu as pltpu

B = 64
P = 576
D = 384
CB = 4
NCHUNK = B // CB
NBUF = 4


def _body(pos_ref, table_ref, patches_hbm, out_hbm,
          emb_ref, inbuf, outbuf, insems, outsems):
    def in_copy(k, slot):
        return pltpu.make_async_copy(
            patches_hbm.at[pl.ds(k * CB, CB)], inbuf.at[slot], insems.at[slot])

    def out_copy(k, slot):
        return pltpu.make_async_copy(
            outbuf.at[slot], out_hbm.at[pl.ds(k * CB, CB)], outsems.at[slot])

    for s in range(NBUF):
        in_copy(s, s).start()

    pos = pos_ref[...]
    onehot = (pos[:, None] == jax.lax.broadcasted_iota(jnp.int32, (P, P), 1))
    emb_ref[...] = jnp.dot(onehot.astype(jnp.float32), table_ref[...],
                           preferred_element_type=jnp.float32)

    for k in range(NCHUNK):
        slot = k % NBUF
        in_copy(k, slot).wait()
        if k >= NBUF:
            out_copy(k - NBUF, slot).wait()
        outbuf[slot] = inbuf[slot] + emb_ref[...][None]
        out_copy(k, slot).start()
        if k + NBUF < NCHUNK:
            in_copy(k + NBUF, slot).start()

    for k in range(NCHUNK - NBUF, NCHUNK):
        out_copy(k, k % NBUF).wait()


_tc_kernel = pl.pallas_call(
    _body,
    in_specs=[
        pl.BlockSpec(memory_space=pltpu.VMEM),
        pl.BlockSpec(memory_space=pltpu.VMEM),
        pl.BlockSpec(memory_space=pl.ANY),
    ],
    out_specs=pl.BlockSpec(memory_space=pl.ANY),
    out_shape=jax.ShapeDtypeStruct((B, P, D), jnp.float32),
    scratch_shapes=[
        pltpu.VMEM((P, D), jnp.float32),
        pltpu.VMEM((NBUF, CB, P, D), jnp.float32),
        pltpu.VMEM((NBUF, CB, P, D), jnp.float32),
        pltpu.SemaphoreType.DMA((NBUF,)),
        pltpu.SemaphoreType.DMA((NBUF,)),
    ],
)


def kernel(encoded_patches, pos_embedding_table, positions):
    return _tc_kernel(positions, pos_embedding_table, encoded_patches)


# speedup vs baseline: 1.0237x; 1.0060x over previous
import jax
import jax.numpy as jnp
from jax.experimental import pallas as pl
from jax.experimental.pallas import tpu as pltpu

B = 64
P = 576
D = 384
CB = 8
NCHUNK = B // CB
NBUF = 3


def _body(pos_ref, table_ref, patches_hbm, out_hbm,
          emb_ref, inbuf, outbuf, insems, outsems):
    def in_copy(k, slot):
        return pltpu.make_async_copy(
            patches_hbm.at[pl.ds(k * CB, CB)], inbuf.at[slot], insems.at[slot])

    def out_copy(k, slot):
        return pltpu.make_async_copy(
            outbuf.at[slot], out_hbm.at[pl.ds(k * CB, CB)], outsems.at[slot])

    for s in range(NBUF):
        in_copy(s, s).start()

    pos = pos_ref[...]
    onehot = (pos[:, None] == jax.lax.broadcasted_iota(jnp.int32, (P, P), 1))
    emb_ref[...] = jnp.dot(onehot.astype(jnp.float32), table_ref[...],
                           preferred_element_type=jnp.float32)

    for k in range(NCHUNK):
        slot = k % NBUF
        in_copy(k, slot).wait()
        if k >= NBUF:
            out_copy(k - NBUF, slot).wait()
        outbuf[slot] = inbuf[slot] + emb_ref[...][None]
        out_copy(k, slot).start()
        if k + NBUF < NCHUNK:
            in_copy(k + NBUF, slot).start()

    for k in range(NCHUNK - NBUF, NCHUNK):
        out_copy(k, k % NBUF).wait()


_tc_kernel = pl.pallas_call(
    _body,
    in_specs=[
        pl.BlockSpec(memory_space=pltpu.VMEM),
        pl.BlockSpec(memory_space=pltpu.VMEM),
        pl.BlockSpec(memory_space=pl.ANY),
    ],
    out_specs=pl.BlockSpec(memory_space=pl.ANY),
    out_shape=jax.ShapeDtypeStruct((B, P, D), jnp.float32),
    scratch_shapes=[
        pltpu.VMEM((P, D), jnp.float32),
        pltpu.VMEM((NBUF, CB, P, D), jnp.float32),
        pltpu.VMEM((NBUF, CB, P, D), jnp.float32),
        pltpu.SemaphoreType.DMA((NBUF,)),
        pltpu.SemaphoreType.DMA((NBUF,)),
    ],
)


def kernel(encoded_patches, pos_embedding_table, positions):
    return _tc_kernel(positions, pos_embedding_table, encoded_patches)


# speedup vs baseline: 1.0749x; 1.0501x over previous
import jax
import jax.numpy as jnp
from jax.experimental import pallas as pl
from jax.experimental.pallas import tpu as pltpu

B = 64
P = 576
D = 384
BB = 16


def _add_body(pos_ref, table_ref, patches_ref, out_ref, emb_ref):
    @pl.when(pl.program_id(0) == 0)
    def _():
        pos = pos_ref[...]
        onehot = (pos[:, None] == jax.lax.broadcasted_iota(jnp.int32, (P, P), 1))
        emb_ref[...] = jnp.dot(onehot.astype(jnp.float32), table_ref[...],
                               preferred_element_type=jnp.float32)

    out_ref[...] = patches_ref[...] + emb_ref[...][None]


_tc_kernel = pl.pallas_call(
    _add_body,
    grid=(B // BB,),
    in_specs=[
        pl.BlockSpec((P,), lambda i: (0,)),
        pl.BlockSpec((P, D), lambda i: (0, 0)),
        pl.BlockSpec((BB, P, D), lambda i: (i, 0, 0)),
    ],
    out_specs=pl.BlockSpec((BB, P, D), lambda i: (i, 0, 0)),
    out_shape=jax.ShapeDtypeStruct((B, P, D), jnp.float32),
    scratch_shapes=[pltpu.VMEM((P, D), jnp.float32)],
)


def kernel(encoded_patches, pos_embedding_table, positions):
    return _tc_kernel(positions, pos_embedding_table, encoded_patches)
